# baseline (speedup 1.0000x reference)
import functools

import jax
import jax.numpy as jnp
from jax import lax
from jax.experimental import pallas as pl
from jax.experimental.pallas import tpu as pltpu
from jax.experimental.pallas import tpu_sc as plsc

T = 2048
D = 768
F = 2048
E = 8
CAP = 256
LANES = 128
NB = 8
TB = T // NB
FBLK = 1024
NFB = F // FBLK
SENTINEL = 1 << 20
NW = 32
TPW = T // NW
CH = 32


def _routing_body(x_ref, wgt_ref, a1_ref, a2_ref, p1_ref, p2_ref, ps_ref,
                  z_ref, h0_ref):
    x = x_ref[...]
    lg = jnp.dot(x, wgt_ref[...], preferred_element_type=jnp.float32)
    lane = jax.lax.broadcasted_iota(jnp.int32, (T, LANES), 1)
    valid = lane < E
    neg = jnp.float32(-jnp.inf)
    lgm = jnp.where(valid, lg, neg)
    m8 = jnp.max(lgm, axis=1, keepdims=True)
    ex = jnp.where(valid, jnp.exp(lgm - m8), 0.0)
    s8 = jnp.sum(ex, axis=1, keepdims=True)
    probs = ex / s8
    ps_ref[...] = jnp.sum(probs, axis=0, keepdims=True)
    lse = m8 + jnp.log(s8)
    z_ref[...] = jnp.sum(lse * lse, axis=0, keepdims=True) * (0.001 / T)
    v1 = m8
    a1 = jnp.min(jnp.where((lgm == v1) & valid, lane, LANES), axis=1, keepdims=True)
    lgm2 = jnp.where(lane == a1, neg, lgm)
    v2 = jnp.max(lgm2, axis=1, keepdims=True)
    a2 = jnp.min(jnp.where(lgm2 == v2, lane, LANES), axis=1, keepdims=True)
    t2 = jnp.exp(v2 - v1)
    s = 1.0 + t2
    q1 = 1.0 / s
    q2 = t2 / s
    ssum = jnp.maximum(q1 + q2, 1e-8)
    a1_ref[...] = a1
    a2_ref[...] = a2
    p1_ref[...] = q1 / ssum
    p2_ref[...] = q2 / ssum
    h0_ref[...] = jnp.sum(jnp.where(lane == a1, 1, 0), axis=0, keepdims=True)


def _rank_body(a1c_ref, a2c_ref, p1c_ref, p2c_ref,
               a1r_ref, a2r_ref, p1r_ref, p2r_ref, ps_ref, h0_ref,
               s1_ref, s2_ref, p1k_ref, p2k_ref,
               s1x_ref, s2x_ref, lb_ref, tpe_scr):
    b = pl.program_id(0)
    ti = b * TB + jax.lax.broadcasted_iota(jnp.int32, (TB, 1), 0)
    tj = jax.lax.broadcasted_iota(jnp.int32, (1, T), 1)
    e1 = a1c_ref[...]
    e2 = a2c_ref[...]
    pc1 = p1c_ref[...]
    pc2 = p2c_ref[...]
    er1 = a1r_ref[...]
    er2 = a2r_ref[...]
    pr1 = p1r_ref[...]
    pr2 = p2r_ref[...]

    def quad(ec, pc, er, pr, before):
        same = er == ec
        beats = (pr > pc) | ((pr == pc) & before)
        return jnp.sum(jnp.where(same & beats, 1, 0), axis=1, keepdims=True)

    r1 = quad(e1, pc1, er1, pr1, tj < ti)
    r2 = quad(e2, pc2, er2, pr2, tj < ti)
    near = (jnp.sum(jnp.where(pr1 < 0.5 + 1e-6, 1, 0))
            + jnp.sum(jnp.where(pr2 > 0.5 - 1e-6, 1, 0))) > 0

    def exact_cross(_):
        c1 = quad(e1, pc1, er2, pr2, tj < ti)
        c2 = quad(e2, pc2, er1, pr1, tj <= ti)
        return c1, c2

    def fast_cross(_):
        c1 = jnp.zeros((TB, 1), jnp.int32)
        lane = jax.lax.broadcasted_iota(jnp.int32, (TB, LANES), 1)
        h0 = jnp.sum(jnp.where(lane == e2, h0_ref[...], 0), axis=1,
                     keepdims=True)
        return c1, h0

    c1, c2 = lax.cond(near, exact_cross, fast_cross, 0)
    r1 = r1 + c1
    r2 = r2 + c2
    keep1 = r1 < CAP
    keep2 = r2 < CAP
    s1 = jnp.where(keep1, e1 * CAP + r1, SENTINEL)
    s2 = jnp.where(keep2, e2 * CAP + r2, SENTINEL)
    p1k = jnp.where(keep1, pc1, 0.0)
    p2k = jnp.where(keep2, pc2, 0.0)
    s1_ref[...] = s1
    s2_ref[...] = s2
    p1k_ref[...] = p1k
    p2k_ref[...] = p2k
    s1x_ref[...] = jnp.where(keep1, s1, T)
    s2x_ref[...] = jnp.where(keep2, s2, T)
    lane = jax.lax.broadcasted_iota(jnp.int32, (TB, LANES), 1)
    oh = jnp.where((lane == e1) & keep1, 1.0, 0.0)
    contrib = jnp.sum(oh, axis=0, keepdims=True)
    acc = jnp.where(b == 0, contrib, tpe_scr[...] + contrib)
    tpe_scr[...] = acc

    @pl.when(b == NB - 1)
    def _():
        lb_ref[...] = jnp.sum(ps_ref[...] * acc, axis=1,
                              keepdims=True) * (0.01 / (T * E))


def _ffn_body(x_ref, w1_ref, w3_ref, w2_ref, s1r_ref, s2r_ref,
              p1r_ref, p2r_ref, y_ref, buf_scr, w_scr):
    e = pl.program_id(0)
    fb = pl.program_id(1)
    bf = jnp.bfloat16

    @pl.when(e == E)
    def _():
        y_ref[...] = jnp.zeros((CAP, D), jnp.float32)

    @pl.when(e < E)
    def _():
        @pl.when(fb == 0)
        def _():
            srow = e * CAP + jax.lax.broadcasted_iota(jnp.int32, (CAP, 1), 0)
            sel = (s1r_ref[...] == srow) | (s2r_ref[...] == srow)
            disp = jnp.where(sel, 1.0, 0.0).astype(bf)
            buf_scr[...] = jnp.dot(disp, x_ref[...].astype(bf),
                                   preferred_element_type=jnp.float32)
            w = (jnp.where(s1r_ref[...] == srow, p1r_ref[...], 0.0)
                 + jnp.where(s2r_ref[...] == srow, p2r_ref[...], 0.0))
            w_scr[...] = jnp.sum(w, axis=1, keepdims=True)

        buf = buf_scr[...].astype(bf)
        nt = (((1,), (1,)), ((), ()))
        h1 = jax.lax.dot_general(buf, w1_ref[0].astype(bf), nt,
                                 preferred_element_type=jnp.float32)
        h3 = jax.lax.dot_general(buf, w3_ref[0].astype(bf), nt,
                                 preferred_element_type=jnp.float32)
        h = (h1 * jax.lax.logistic(h1) * h3).astype(bf)
        yp = jax.lax.dot_general(h, w2_ref[0].astype(bf), nt,
                                 preferred_element_type=jnp.float32)

        @pl.when(fb == 0)
        def _():
            y_ref[...] = yp

        @pl.when((fb > 0) & (fb < NFB - 1))
        def _():
            y_ref[...] = y_ref[...] + yp

        @pl.when(fb == NFB - 1)
        def _():
            y_ref[...] = (y_ref[...] + yp) * w_scr[...]


def _sc_combine(y, s1x, s2x):
    mesh = plsc.VectorSubcoreMesh(core_axis_name="c", subcore_axis_name="s")

    @functools.partial(
        pl.kernel, mesh=mesh,
        out_type=jax.ShapeDtypeStruct((T, D), jnp.float32),
        scratch_types=[
            pltpu.VMEM((CH,), jnp.int32),
            pltpu.VMEM((CH,), jnp.int32),
            pltpu.VMEM((CH,), jnp.int32),
            pltpu.VMEM((CH, D), jnp.float32),
            pltpu.VMEM((CH, D), jnp.float32),
            pltpu.VMEM_SHARED((16 * CH, D), jnp.float32),
            pltpu.SemaphoreType.DMA,
        ],
    )
    def k(y_hbm, s1_hbm, s2_hbm, out_hbm,
          idx0_v, idx1_v, idx2_v, r1_v, r2_v, sh, sem):
        sid = lax.axis_index("s")
        wid = sid * 2 + lax.axis_index("c")
        for c in range(CH // 16):
            idx0_v[pl.ds(c * 16, 16)] = (jax.lax.iota(jnp.int32, 16)
                                         + (c * 16 + sid * CH))
        for ch in range(TPW // CH):
            base = wid * TPW + ch * CH
            pltpu.sync_copy(s1_hbm.at[pl.ds(base, CH)], idx1_v)
            pltpu.sync_copy(s2_hbm.at[pl.ds(base, CH)], idx2_v)
            cp1 = pltpu.async_copy(y_hbm.at[idx1_v], r1_v, sem)
            cp2 = pltpu.async_copy(y_hbm.at[idx2_v], r2_v, sem)
            cp1.wait()
            cp2.wait()
            pltpu.sync_copy(r1_v, out_hbm.at[pl.ds(base, CH)])

    return k(y, s1x, s2x)


def kernel(x, Wg, W1, W3, W2):
    wgt = jnp.zeros((D, LANES), jnp.float32).at[:, :E].set(Wg.T)

    a1, a2, p1, p2, ps, z, h0 = pl.pallas_call(
        _routing_body,
        out_shape=(
            jax.ShapeDtypeStruct((T, 1), jnp.int32),
            jax.ShapeDtypeStruct((T, 1), jnp.int32),
            jax.ShapeDtypeStruct((T, 1), jnp.float32),
            jax.ShapeDtypeStruct((T, 1), jnp.float32),
            jax.ShapeDtypeStruct((1, LANES), jnp.float32),
            jax.ShapeDtypeStruct((1, 1), jnp.float32),
            jax.ShapeDtypeStruct((1, LANES), jnp.int32),
        ),
    )(x, wgt)

    a1r = a1.reshape(1, T)
    a2r = a2.reshape(1, T)
    p1r = p1.reshape(1, T)
    p2r = p2.reshape(1, T)

    col = pl.BlockSpec((TB, 1), lambda b: (b, 0))
    row = pl.BlockSpec((1, T), lambda b: (0, 0))
    one = pl.BlockSpec((1, 1), lambda b: (0, 0))
    vec = pl.BlockSpec((1, LANES), lambda b: (0, 0))
    s1, s2, p1k, p2k, s1x, s2x, lb = pl.pallas_call(
        _rank_body,
        grid=(NB,),
        in_specs=[col, col, col, col, row, row, row, row, vec, vec],
        out_specs=(col, col, col, col, col, col, one),
        out_shape=(
            jax.ShapeDtypeStruct((T, 1), jnp.int32),
            jax.ShapeDtypeStruct((T, 1), jnp.int32),
            jax.ShapeDtypeStruct((T, 1), jnp.float32),
            jax.ShapeDtypeStruct((T, 1), jnp.float32),
            jax.ShapeDtypeStruct((T, 1), jnp.int32),
            jax.ShapeDtypeStruct((T, 1), jnp.int32),
            jax.ShapeDtypeStruct((1, 1), jnp.float32),
        ),
        scratch_shapes=[pltpu.VMEM((1, LANES), jnp.float32)],
    )(a1, a2, p1, p2, a1r, a2r, p1r, p2r, ps, h0)

    s1r = s1.reshape(1, T)
    s2r = s2.reshape(1, T)
    p1kr = p1k.reshape(1, T)
    p2kr = p2k.reshape(1, T)

    wclamp = lambda e, f: (jnp.minimum(e, E - 1), f, 0)
    w2clamp = lambda e, f: (jnp.minimum(e, E - 1), 0, f)
    y = pl.pallas_call(
        _ffn_body,
        grid=(E + 1, NFB),
        in_specs=[
            pl.BlockSpec((T, D), lambda e, f: (0, 0)),
            pl.BlockSpec((1, FBLK, D), wclamp),
            pl.BlockSpec((1, FBLK, D), wclamp),
            pl.BlockSpec((1, D, FBLK), w2clamp),
            pl.BlockSpec((1, T), lambda e, f: (0, 0)),
            pl.BlockSpec((1, T), lambda e, f: (0, 0)),
            pl.BlockSpec((1, T), lambda e, f: (0, 0)),
            pl.BlockSpec((1, T), lambda e, f: (0, 0)),
        ],
        out_specs=pl.BlockSpec((CAP, D), lambda e, f: (e, 0)),
        out_shape=jax.ShapeDtypeStruct((T + CAP, D), jnp.float32),
        scratch_shapes=[pltpu.VMEM((CAP, D), jnp.float32),
                        pltpu.VMEM((CAP, 1), jnp.float32)],
    )(x, W1, W3, W2, s1r, s2r, p1kr, p2kr)

    out = _sc_combine(y, s1x.reshape(T), s2x.reshape(T))
    return out, lb.reshape(()), z.reshape(())


# speedup vs baseline: 2.1491x; 2.1491x over previous
import jax
import jax.numpy as jnp
from jax import lax
from jax.experimental import pallas as pl
from jax.experimental.pallas import tpu as pltpu

T = 2048
D = 768
F = 2048
E = 8
CAP = 256
LANES = 128
NB = 8
TB = T // NB
FBLK = 1024
NFB = F // FBLK
SENTINEL = 1 << 20


def _routing_body(x_ref, wgt_ref, a1_ref, a2_ref, p1_ref, p2_ref, ps_ref,
                  z_ref, h0_ref):
    x = x_ref[...]
    lg = jnp.dot(x, wgt_ref[...], preferred_element_type=jnp.float32)
    lane = jax.lax.broadcasted_iota(jnp.int32, (T, LANES), 1)
    valid = lane < E
    neg = jnp.float32(-jnp.inf)
    lgm = jnp.where(valid, lg, neg)
    m8 = jnp.max(lgm, axis=1, keepdims=True)
    ex = jnp.where(valid, jnp.exp(lgm - m8), 0.0)
    s8 = jnp.sum(ex, axis=1, keepdims=True)
    probs = ex / s8
    ps_ref[...] = jnp.sum(probs, axis=0, keepdims=True)
    lse = m8 + jnp.log(s8)
    z_ref[...] = jnp.sum(lse * lse, axis=0, keepdims=True) * (0.001 / T)
    v1 = m8
    a1 = jnp.min(jnp.where((lgm == v1) & valid, lane, LANES), axis=1, keepdims=True)
    lgm2 = jnp.where(lane == a1, neg, lgm)
    v2 = jnp.max(lgm2, axis=1, keepdims=True)
    a2 = jnp.min(jnp.where(lgm2 == v2, lane, LANES), axis=1, keepdims=True)
    t2 = jnp.exp(v2 - v1)
    s = 1.0 + t2
    q1 = 1.0 / s
    q2 = t2 / s
    ssum = jnp.maximum(q1 + q2, 1e-8)
    a1_ref[...] = a1
    a2_ref[...] = a2
    p1_ref[...] = q1 / ssum
    p2_ref[...] = q2 / ssum
    h0_ref[...] = jnp.sum(jnp.where(lane == a1, 1, 0), axis=0, keepdims=True)


def _rank_body(a1c_ref, a2c_ref, p1c_ref, p2c_ref,
               a1r_ref, a2r_ref, p1r_ref, p2r_ref, ps_ref, h0_ref,
               s1_ref, s2_ref, p1k_ref, p2k_ref, lb_ref, tpe_scr):
    b = pl.program_id(0)
    ti = b * TB + jax.lax.broadcasted_iota(jnp.int32, (TB, 1), 0)
    tj = jax.lax.broadcasted_iota(jnp.int32, (1, T), 1)
    e1 = a1c_ref[...]
    e2 = a2c_ref[...]
    pc1 = p1c_ref[...]
    pc2 = p2c_ref[...]
    er1 = a1r_ref[...]
    er2 = a2r_ref[...]
    pr1 = p1r_ref[...]
    pr2 = p2r_ref[...]

    def quad(ec, pc, er, pr, before):
        same = er == ec
        beats = (pr > pc) | ((pr == pc) & before)
        return jnp.sum(jnp.where(same & beats, 1, 0), axis=1, keepdims=True)

    r1 = quad(e1, pc1, er1, pr1, tj < ti)
    r2 = quad(e2, pc2, er2, pr2, tj < ti)
    near = (jnp.sum(jnp.where(pr1 < 0.5 + 1e-6, 1, 0))
            + jnp.sum(jnp.where(pr2 > 0.5 - 1e-6, 1, 0))) > 0

    def exact_cross(_):
        c1 = quad(e1, pc1, er2, pr2, tj < ti)
        c2 = quad(e2, pc2, er1, pr1, tj <= ti)
        return c1, c2

    def fast_cross(_):
        c1 = jnp.zeros((TB, 1), jnp.int32)
        lane = jax.lax.broadcasted_iota(jnp.int32, (TB, LANES), 1)
        h0 = jnp.sum(jnp.where(lane == e2, h0_ref[...], 0), axis=1,
                     keepdims=True)
        return c1, h0

    c1, c2 = lax.cond(near, exact_cross, fast_cross, 0)
    r1 = r1 + c1
    r2 = r2 + c2
    keep1 = r1 < CAP
    keep2 = r2 < CAP
    s1 = jnp.where(keep1, e1 * CAP + r1, SENTINEL)
    s2 = jnp.where(keep2, e2 * CAP + r2, SENTINEL)
    p1k = jnp.where(keep1, pc1, 0.0)
    p2k = jnp.where(keep2, pc2, 0.0)
    s1_ref[...] = s1
    s2_ref[...] = s2
    p1k_ref[...] = p1k
    p2k_ref[...] = p2k
    lane = jax.lax.broadcasted_iota(jnp.int32, (TB, LANES), 1)
    oh = jnp.where((lane == e1) & keep1, 1.0, 0.0)
    contrib = jnp.sum(oh, axis=0, keepdims=True)
    acc = jnp.where(b == 0, contrib, tpe_scr[...] + contrib)
    tpe_scr[...] = acc

    @pl.when(b == NB - 1)
    def _():
        lb_ref[...] = jnp.sum(ps_ref[...] * acc, axis=1,
                              keepdims=True) * (0.01 / (T * E))


def _ffn_body(x_ref, w1_ref, w3_ref, w2_ref,
              s1r_ref, s2r_ref, s1c_ref, s2c_ref, p1k_ref, p2k_ref,
              out_ref, buf_scr, yacc_scr):
    e = pl.program_id(0)
    fb = pl.program_id(1)
    bf = jnp.bfloat16

    @pl.when(fb == 0)
    def _():
        srow = e * CAP + jax.lax.broadcasted_iota(jnp.int32, (CAP, 1), 0)
        sel = (s1r_ref[...] == srow) | (s2r_ref[...] == srow)
        disp = jnp.where(sel, 1.0, 0.0).astype(bf)
        buf_scr[...] = jnp.dot(disp, x_ref[...].astype(bf),
                               preferred_element_type=jnp.float32)

    buf = buf_scr[...].astype(bf)
    nt = (((1,), (1,)), ((), ()))
    h1 = jax.lax.dot_general(buf, w1_ref[0].astype(bf), nt,
                             preferred_element_type=jnp.float32)
    h3 = jax.lax.dot_general(buf, w3_ref[0].astype(bf), nt,
                             preferred_element_type=jnp.float32)
    h = (h1 * jax.lax.logistic(h1) * h3).astype(bf)
    yp = jax.lax.dot_general(h, w2_ref[0].astype(bf), nt,
                             preferred_element_type=jnp.float32)

    @pl.when(fb == 0)
    def _():
        yacc_scr[...] = yp

    @pl.when(fb > 0)
    def _():
        yacc_scr[...] = yacc_scr[...] + yp

    @pl.when(fb == NFB - 1)
    def _():
        crow = e * CAP + jax.lax.broadcasted_iota(jnp.int32, (1, CAP), 1)
        comb = (jnp.where(s1c_ref[...] == crow, p1k_ref[...], 0.0)
                + jnp.where(s2c_ref[...] == crow, p2k_ref[...], 0.0))
        contrib = jnp.dot(comb.astype(bf), yacc_scr[...].astype(bf),
                          preferred_element_type=jnp.float32)

        @pl.when(e == 0)
        def _():
            out_ref[...] = contrib

        @pl.when(e > 0)
        def _():
            out_ref[...] = out_ref[...] + contrib


def kernel(x, Wg, W1, W3, W2):
    wgt = jnp.zeros((D, LANES), jnp.float32).at[:, :E].set(Wg.T)

    a1, a2, p1, p2, ps, z, h0 = pl.pallas_call(
        _routing_body,
        out_shape=(
            jax.ShapeDtypeStruct((T, 1), jnp.int32),
            jax.ShapeDtypeStruct((T, 1), jnp.int32),
            jax.ShapeDtypeStruct((T, 1), jnp.float32),
            jax.ShapeDtypeStruct((T, 1), jnp.float32),
            jax.ShapeDtypeStruct((1, LANES), jnp.float32),
            jax.ShapeDtypeStruct((1, 1), jnp.float32),
            jax.ShapeDtypeStruct((1, LANES), jnp.int32),
        ),
    )(x, wgt)

    a1r = a1.reshape(1, T)
    a2r = a2.reshape(1, T)
    p1r = p1.reshape(1, T)
    p2r = p2.reshape(1, T)

    col = pl.BlockSpec((TB, 1), lambda b: (b, 0))
    row = pl.BlockSpec((1, T), lambda b: (0, 0))
    one = pl.BlockSpec((1, 1), lambda b: (0, 0))
    vec = pl.BlockSpec((1, LANES), lambda b: (0, 0))
    s1, s2, p1k, p2k, lb = pl.pallas_call(
        _rank_body,
        grid=(NB,),
        in_specs=[col, col, col, col, row, row, row, row, vec, vec],
        out_specs=(col, col, col, col, one),
        out_shape=(
            jax.ShapeDtypeStruct((T, 1), jnp.int32),
            jax.ShapeDtypeStruct((T, 1), jnp.int32),
            jax.ShapeDtypeStruct((T, 1), jnp.float32),
            jax.ShapeDtypeStruct((T, 1), jnp.float32),
            jax.ShapeDtypeStruct((1, 1), jnp.float32),
        ),
        scratch_shapes=[pltpu.VMEM((1, LANES), jnp.float32)],
    )(a1, a2, p1, p2, a1r, a2r, p1r, p2r, ps, h0)

    s1r = s1.reshape(1, T)
    s2r = s2.reshape(1, T)

    out = pl.pallas_call(
        _ffn_body,
        grid=(E, NFB),
        in_specs=[
            pl.BlockSpec((T, D), lambda e, f: (0, 0)),
            pl.BlockSpec((1, FBLK, D), lambda e, f: (e, f, 0)),
            pl.BlockSpec((1, FBLK, D), lambda e, f: (e, f, 0)),
            pl.BlockSpec((1, D, FBLK), lambda e, f: (e, 0, f)),
            pl.BlockSpec((1, T), lambda e, f: (0, 0)),
            pl.BlockSpec((1, T), lambda e, f: (0, 0)),
            pl.BlockSpec((T, 1), lambda e, f: (0, 0)),
            pl.BlockSpec((T, 1), lambda e, f: (0, 0)),
            pl.BlockSpec((T, 1), lambda e, f: (0, 0)),
            pl.BlockSpec((T, 1), lambda e, f: (0, 0)),
        ],
        out_specs=pl.BlockSpec((T, D), lambda e, f: (0, 0)),
        out_shape=jax.ShapeDtypeStruct((T, D), jnp.float32),
        scratch_shapes=[pltpu.VMEM((CAP, D), jnp.float32),
                        pltpu.VMEM((CAP, D), jnp.float32)],
    )(x, W1, W3, W2, s1r, s2r, s1, s2, p1k, p2k)

    return out, lb.reshape(()), z.reshape(())


# speedup vs baseline: 2.3684x; 1.1020x over previous
import jax
import jax.numpy as jnp
from jax import lax
from jax.experimental import pallas as pl
from jax.experimental.pallas import tpu as pltpu

T = 2048
D = 768
F = 2048
E = 8
CAP = 256
LANES = 128
NB = 8
TB = T // NB
FBLK = 2048
NFB = F // FBLK
SENTINEL = 1 << 20


def _routing_body(x_ref, wgt_ref, a1_ref, a2_ref, p1_ref, p2_ref, ps_ref,
                  z_ref, h0_ref):
    x = x_ref[...]
    lg = jnp.dot(x, wgt_ref[...], preferred_element_type=jnp.float32)
    lane = jax.lax.broadcasted_iota(jnp.int32, (T, LANES), 1)
    valid = lane < E
    neg = jnp.float32(-jnp.inf)
    lgm = jnp.where(valid, lg, neg)
    m8 = jnp.max(lgm, axis=1, keepdims=True)
    ex = jnp.where(valid, jnp.exp(lgm - m8), 0.0)
    s8 = jnp.sum(ex, axis=1, keepdims=True)
    probs = ex / s8
    ps_ref[...] = jnp.sum(probs, axis=0, keepdims=True)
    lse = m8 + jnp.log(s8)
    z_ref[...] = jnp.sum(lse * lse, axis=0, keepdims=True) * (0.001 / T)
    v1 = m8
    a1 = jnp.min(jnp.where((lgm == v1) & valid, lane, LANES), axis=1, keepdims=True)
    lgm2 = jnp.where(lane == a1, neg, lgm)
    v2 = jnp.max(lgm2, axis=1, keepdims=True)
    a2 = jnp.min(jnp.where(lgm2 == v2, lane, LANES), axis=1, keepdims=True)
    t2 = jnp.exp(v2 - v1)
    s = 1.0 + t2
    q1 = 1.0 / s
    q2 = t2 / s
    ssum = jnp.maximum(q1 + q2, 1e-8)
    a1_ref[...] = a1
    a2_ref[...] = a2
    p1_ref[...] = q1 / ssum
    p2_ref[...] = q2 / ssum
    h0_ref[...] = jnp.sum(jnp.where(lane == a1, 1, 0), axis=0, keepdims=True)


def _rank_body(a1c_ref, a2c_ref, p1c_ref, p2c_ref,
               a1r_ref, a2r_ref, p1r_ref, p2r_ref, ps_ref, h0_ref,
               s1_ref, s2_ref, p1k_ref, p2k_ref, lb_ref, tpe_scr):
    b = pl.program_id(0)
    ti = b * TB + jax.lax.broadcasted_iota(jnp.int32, (TB, 1), 0)
    tj = jax.lax.broadcasted_iota(jnp.int32, (1, T), 1)
    e1 = a1c_ref[...]
    e2 = a2c_ref[...]
    pc1 = p1c_ref[...]
    pc2 = p2c_ref[...]
    er1 = a1r_ref[...]
    er2 = a2r_ref[...]
    pr1 = p1r_ref[...]
    pr2 = p2r_ref[...]

    def quad(ec, pc, er, pr, before):
        same = er == ec
        beats = (pr > pc) | ((pr == pc) & before)
        return jnp.sum(jnp.where(same & beats, 1, 0), axis=1, keepdims=True)

    r1 = quad(e1, pc1, er1, pr1, tj < ti)
    r2 = quad(e2, pc2, er2, pr2, tj < ti)
    near = (jnp.sum(jnp.where(pr1 < 0.5 + 1e-6, 1, 0))
            + jnp.sum(jnp.where(pr2 > 0.5 - 1e-6, 1, 0))) > 0

    def exact_cross(_):
        c1 = quad(e1, pc1, er2, pr2, tj < ti)
        c2 = quad(e2, pc2, er1, pr1, tj <= ti)
        return c1, c2

    def fast_cross(_):
        c1 = jnp.zeros((TB, 1), jnp.int32)
        lane = jax.lax.broadcasted_iota(jnp.int32, (TB, LANES), 1)
        h0 = jnp.sum(jnp.where(lane == e2, h0_ref[...], 0), axis=1,
                     keepdims=True)
        return c1, h0

    c1, c2 = lax.cond(near, exact_cross, fast_cross, 0)
    r1 = r1 + c1
    r2 = r2 + c2
    keep1 = r1 < CAP
    keep2 = r2 < CAP
    s1 = jnp.where(keep1, e1 * CAP + r1, SENTINEL)
    s2 = jnp.where(keep2, e2 * CAP + r2, SENTINEL)
    p1k = jnp.where(keep1, pc1, 0.0)
    p2k = jnp.where(keep2, pc2, 0.0)
    s1_ref[...] = s1
    s2_ref[...] = s2
    p1k_ref[...] = p1k
    p2k_ref[...] = p2k
    lane = jax.lax.broadcasted_iota(jnp.int32, (TB, LANES), 1)
    oh = jnp.where((lane == e1) & keep1, 1.0, 0.0)
    contrib = jnp.sum(oh, axis=0, keepdims=True)
    acc = jnp.where(b == 0, contrib, tpe_scr[...] + contrib)
    tpe_scr[...] = acc

    @pl.when(b == NB - 1)
    def _():
        lb_ref[...] = jnp.sum(ps_ref[...] * acc, axis=1,
                              keepdims=True) * (0.01 / (T * E))


def _ffn_body(x_ref, w1_ref, w3_ref, w2_ref,
              s1r_ref, s2r_ref, s1c_ref, s2c_ref, p1k_ref, p2k_ref,
              out_ref, buf_scr, yacc_scr):
    e = pl.program_id(0)
    fb = pl.program_id(1)
    bf = jnp.bfloat16

    @pl.when(fb == 0)
    def _():
        srow = e * CAP + jax.lax.broadcasted_iota(jnp.int32, (CAP, 1), 0)
        sel = (s1r_ref[...] == srow) | (s2r_ref[...] == srow)
        disp = jnp.where(sel, 1.0, 0.0).astype(bf)
        buf_scr[...] = jnp.dot(disp, x_ref[...].astype(bf),
                               preferred_element_type=jnp.float32)

    buf = buf_scr[...].astype(bf)
    nt = (((1,), (1,)), ((), ()))
    h1 = jax.lax.dot_general(buf, w1_ref[0].astype(bf), nt,
                             preferred_element_type=jnp.float32)
    h3 = jax.lax.dot_general(buf, w3_ref[0].astype(bf), nt,
                             preferred_element_type=jnp.float32)
    h = (h1 * jax.lax.logistic(h1) * h3).astype(bf)
    yp = jax.lax.dot_general(h, w2_ref[0].astype(bf), nt,
                             preferred_element_type=jnp.float32)

    @pl.when(fb == 0)
    def _():
        yacc_scr[...] = yp

    @pl.when(fb > 0)
    def _():
        yacc_scr[...] = yacc_scr[...] + yp

    @pl.when(fb == NFB - 1)
    def _():
        crow = e * CAP + jax.lax.broadcasted_iota(jnp.int32, (1, CAP), 1)
        comb = (jnp.where(s1c_ref[...] == crow, p1k_ref[...], 0.0)
                + jnp.where(s2c_ref[...] == crow, p2k_ref[...], 0.0))
        contrib = jnp.dot(comb.astype(bf), yacc_scr[...].astype(bf),
                          preferred_element_type=jnp.float32)

        @pl.when(e == 0)
        def _():
            out_ref[...] = contrib

        @pl.when(e > 0)
        def _():
            out_ref[...] = out_ref[...] + contrib


def kernel(x, Wg, W1, W3, W2):
    wgt = jnp.zeros((D, LANES), jnp.float32).at[:, :E].set(Wg.T)

    a1, a2, p1, p2, ps, z, h0 = pl.pallas_call(
        _routing_body,
        out_shape=(
            jax.ShapeDtypeStruct((T, 1), jnp.int32),
            jax.ShapeDtypeStruct((T, 1), jnp.int32),
            jax.ShapeDtypeStruct((T, 1), jnp.float32),
            jax.ShapeDtypeStruct((T, 1), jnp.float32),
            jax.ShapeDtypeStruct((1, LANES), jnp.float32),
            jax.ShapeDtypeStruct((1, 1), jnp.float32),
            jax.ShapeDtypeStruct((1, LANES), jnp.int32),
        ),
    )(x, wgt)

    a1r = a1.reshape(1, T)
    a2r = a2.reshape(1, T)
    p1r = p1.reshape(1, T)
    p2r = p2.reshape(1, T)

    col = pl.BlockSpec((TB, 1), lambda b: (b, 0))
    row = pl.BlockSpec((1, T), lambda b: (0, 0))
    one = pl.BlockSpec((1, 1), lambda b: (0, 0))
    vec = pl.BlockSpec((1, LANES), lambda b: (0, 0))
    s1, s2, p1k, p2k, lb = pl.pallas_call(
        _rank_body,
        grid=(NB,),
        in_specs=[col, col, col, col, row, row, row, row, vec, vec],
        out_specs=(col, col, col, col, one),
        out_shape=(
            jax.ShapeDtypeStruct((T, 1), jnp.int32),
            jax.ShapeDtypeStruct((T, 1), jnp.int32),
            jax.ShapeDtypeStruct((T, 1), jnp.float32),
            jax.ShapeDtypeStruct((T, 1), jnp.float32),
            jax.ShapeDtypeStruct((1, 1), jnp.float32),
        ),
        scratch_shapes=[pltpu.VMEM((1, LANES), jnp.float32)],
    )(a1, a2, p1, p2, a1r, a2r, p1r, p2r, ps, h0)

    s1r = s1.reshape(1, T)
    s2r = s2.reshape(1, T)

    out = pl.pallas_call(
        _ffn_body,
        grid=(E, NFB),
        in_specs=[
            pl.BlockSpec((T, D), lambda e, f: (0, 0)),
            pl.BlockSpec((1, FBLK, D), lambda e, f: (e, f, 0)),
            pl.BlockSpec((1, FBLK, D), lambda e, f: (e, f, 0)),
            pl.BlockSpec((1, D, FBLK), lambda e, f: (e, 0, f)),
            pl.BlockSpec((1, T), lambda e, f: (0, 0)),
            pl.BlockSpec((1, T), lambda e, f: (0, 0)),
            pl.BlockSpec((T, 1), lambda e, f: (0, 0)),
            pl.BlockSpec((T, 1), lambda e, f: (0, 0)),
            pl.BlockSpec((T, 1), lambda e, f: (0, 0)),
            pl.BlockSpec((T, 1), lambda e, f: (0, 0)),
        ],
        out_specs=pl.BlockSpec((T, D), lambda e, f: (0, 0)),
        out_shape=jax.ShapeDtypeStruct((T, D), jnp.float32),
        scratch_shapes=[pltpu.VMEM((CAP, D), jnp.float32),
                        pltpu.VMEM((CAP, D), jnp.float32)],
    )(x, W1, W3, W2, s1r, s2r, s1, s2, p1k, p2k)

    return out, lb.reshape(()), z.reshape(())
